# baseline (speedup 1.0000x reference)
import jax
import jax.numpy as jnp
from jax.experimental import pallas as pl
from jax.experimental.pallas import tpu as pltpu

_B = 4
_NC = 16
_IMG = 320
_HM = 32
_WM = 32
_K = 256
_ACS = 8
_SLOPE = 5.0
_KH = 5
_KW = 5
_C = 2 * _NC
_TH = 32
_NS = _IMG // _TH
_HPAD = _IMG + 4
_KDIM = _KH * _KW * _C
_RU = 8


def _mask_kernel(mi_ref, cw_ref, bm_ref, adj_ref):
    w = cw_ref[...]
    col = jax.lax.broadcasted_iota(jnp.int32, (_HM, _WM), 1)
    acs_lo = _WM // 2 - _ACS // 2
    acs_hi = _WM // 2 + _ACS // 2 + 1
    w = jnp.where((col >= acs_lo) & (col < acs_hi), jnp.float32(1.0e7), w)
    mi = mi_ref[...].reshape(_B, 1, 1)
    sig = jax.nn.sigmoid(mi * w[None, :, :])
    bits = jax.lax.bitcast_convert_type(sig, jnp.int32)

    def step(_, lohi):
        lo, hi = lohi
        mid = lo + (hi - lo + 1) // 2
        cnt = jnp.sum((bits >= mid).astype(jnp.int32), axis=(1, 2),
                      keepdims=True)
        ge = cnt >= _K
        return jnp.where(ge, mid, lo), jnp.where(ge, hi, mid - 1)

    lo0 = jnp.zeros((_B, 1, 1), jnp.int32)
    hi0 = jnp.full((_B, 1, 1), 0x7F800000, jnp.int32)
    lo, _ = jax.lax.fori_loop(0, 31, step, (lo0, hi0))
    comp = bits >= lo
    wta = jnp.where(comp, sig, jnp.float32(0.0))
    p = jax.nn.sigmoid(_SLOPE * wta)
    hard = (p > 0.5).astype(jnp.float32)
    bm = p + (hard - p)
    bm_ref[...] = bm.reshape(_B, 1, _HM, _WM)
    reps_w = _IMG // _WM
    reps_h = _IMG // _HM
    a = jnp.broadcast_to(bm[:, :, None, :], (_B, _HM, reps_w, _WM))
    a = a.reshape(_B, _HM, _IMG)
    adj = jnp.broadcast_to(a[:, None, :, :], (_B, reps_h, _HM, _IMG))
    adj_ref[...] = adj.reshape(_B, _IMG, _IMG)


def _conv_kernel(xT_ref, adj_ref, wl1_ref, wl2_ref, out_ref,
                 mx, k1, xbig, oslab, insem, outsem):
    b = pl.program_id(0)
    cp_in = pltpu.make_async_copy(xT_ref.at[b], mx.at[pl.ds(2, _IMG)], insem)
    cp_in.start()
    zrow = jnp.zeros((2, _C, _IMG), jnp.bfloat16)
    mx[0:2] = zrow
    mx[_IMG + 2:_HPAD] = zrow
    k1[0:2] = zrow
    k1[_IMG + 2:_HPAD] = zrow
    cp_in.wait()

    def mask_chunk(i, _):
        rows = adj_ref[0, pl.ds(i * _TH, _TH), :].astype(jnp.bfloat16)
        mx[pl.ds(2 + i * _TH, _TH)] = (
            mx[pl.ds(2 + i * _TH, _TH)] * rows[:, None, :])
        return 0
    jax.lax.fori_loop(0, _NS, mask_chunk, 0)

    def conv_pass(src, wl_ref, write_row):
        wl = wl_ref[...]

        def slab_body(s, _):
            h0 = s * _TH
            for dw in range(_KW):
                sh = 2 - dw
                c0 = dw * _C
                if sh > 0:
                    xbig[:, c0:c0 + _C, sh:] = (
                        src[pl.ds(h0, _TH + 4), :, :_IMG - sh])
                    xbig[:, c0:c0 + _C, :sh] = jnp.zeros(
                        (_TH + 4, _C, sh), jnp.bfloat16)
                elif sh < 0:
                    xbig[:, c0:c0 + _C, :sh] = (
                        src[pl.ds(h0, _TH + 4), :, -sh:])
                    xbig[:, c0:c0 + _C, sh:] = jnp.zeros(
                        (_TH + 4, _C, -sh), jnp.bfloat16)
                else:
                    xbig[:, c0:c0 + _C, :] = src[pl.ds(h0, _TH + 4)]

            def row_body(rq, _):
                for k in range(_RU):
                    r = rq * _RU + k
                    xs = xbig[pl.ds(r, _KH)].reshape(_KDIM, _IMG)
                    y = jax.lax.dot_general(
                        wl, xs, (((1,), (0,)), ((), ())),
                        preferred_element_type=jnp.float32)
                    h = h0 + r
                    mrow = adj_ref[0, h, :]
                    write_row(h, r, mrow, src[h + 2], y)
                return 0
            jax.lax.fori_loop(0, _TH // _RU, row_body, 0)

            if write_row is _write_k2:
                cp_out = pltpu.make_async_copy(
                    oslab, out_ref.at[b, pl.ds(h0, _TH)], outsem)
                cp_out.start()
                cp_out.wait()
            return 0
        jax.lax.fori_loop(0, _NS, slab_body, 0)

    def _write_k1(h, r, mrow, mxrow, y):
        k1[h + 2] = jnp.where(
            mrow[None, :] == 1.0, mxrow, y.astype(jnp.bfloat16))

    def _write_k2(h, r, mrow, mxrow, y):
        oslab[r] = jnp.where(
            mrow[None, :] == 1.0, mxrow, y.astype(jnp.bfloat16))

    conv_pass(mx, wl1_ref, _write_k1)
    conv_pass(k1, wl2_ref, _write_k2)


def _build_wl(wr, wi):
    wrz = wr.at[:, :, _KH // 2, _KW // 2].set(0.0)
    wiz = wi.at[:, :, _KH // 2, _KW // 2].set(0.0)
    top = jnp.concatenate([wrz, -wiz], axis=1)
    bot = jnp.concatenate([wiz, wrz], axis=1)
    wfull = jnp.concatenate([top, bot], axis=0)
    return wfull.transpose(0, 2, 3, 1).reshape(_C, _KDIM)


def kernel(x, my_input_1, conv_mask_w, wr0, wi0, wr1, wi1):
    bm, adj = pl.pallas_call(
        _mask_kernel,
        out_shape=(
            jax.ShapeDtypeStruct((_B, 1, _HM, _WM), jnp.float32),
            jax.ShapeDtypeStruct((_B, _IMG, _IMG), jnp.float32),
        ),
    )(my_input_1.reshape(_B, 1), conv_mask_w.reshape(_HM, _WM))

    wl1 = _build_wl(wr0, wi0).astype(jnp.bfloat16)
    wl2 = _build_wl(wr1, wi1).astype(jnp.bfloat16)
    xc = jnp.concatenate([x[..., 0], x[..., 1]], axis=1)
    xT = xc.transpose(0, 2, 1, 3).astype(jnp.bfloat16)

    k2T = pl.pallas_call(
        _conv_kernel,
        grid=(_B,),
        in_specs=[
            pl.BlockSpec(memory_space=pl.ANY),
            pl.BlockSpec((1, _IMG, _IMG), lambda b: (b, 0, 0)),
            pl.BlockSpec((_C, _KDIM), lambda b: (0, 0)),
            pl.BlockSpec((_C, _KDIM), lambda b: (0, 0)),
        ],
        out_specs=pl.BlockSpec(memory_space=pl.ANY),
        out_shape=jax.ShapeDtypeStruct((_B, _IMG, _C, _IMG), jnp.bfloat16),
        scratch_shapes=[
            pltpu.VMEM((_HPAD, _C, _IMG), jnp.bfloat16),
            pltpu.VMEM((_HPAD, _C, _IMG), jnp.bfloat16),
            pltpu.VMEM((_TH + 4, _KW * _C, _IMG), jnp.bfloat16),
            pltpu.VMEM((_TH, _C, _IMG), jnp.bfloat16),
            pltpu.SemaphoreType.DMA,
            pltpu.SemaphoreType.DMA,
        ],
    )(xT, adj, wl1, wl2)

    kc = k2T.transpose(0, 2, 1, 3).astype(jnp.float32)
    kspace_pred = jnp.stack([kc[:, :_NC], kc[:, _NC:]], axis=-1)
    return kspace_pred, adj.reshape(_B, 1, _IMG, _IMG), bm


# speedup vs baseline: 1.0018x; 1.0018x over previous
import jax
import jax.numpy as jnp
from jax.experimental import pallas as pl
from jax.experimental.pallas import tpu as pltpu

_B = 4
_NC = 16
_IMG = 320
_HM = 32
_WM = 32
_K = 256
_ACS = 8
_SLOPE = 5.0
_KH = 5
_KW = 5
_C = 2 * _NC
_TH = 32
_NS = _IMG // _TH
_HPAD = _IMG + 4
_KDIM = _KH * _KW * _C
_RU = 8


def _mask_kernel(mi_ref, cw_ref, bm_ref, adj_ref):
    w = cw_ref[...]
    col = jax.lax.broadcasted_iota(jnp.int32, (_HM, _WM), 1)
    acs_lo = _WM // 2 - _ACS // 2
    acs_hi = _WM // 2 + _ACS // 2 + 1
    w = jnp.where((col >= acs_lo) & (col < acs_hi), jnp.float32(1.0e7), w)
    mi = mi_ref[...].reshape(_B, 1, 1)
    sig = jax.nn.sigmoid(mi * w[None, :, :])
    bits = jax.lax.bitcast_convert_type(sig, jnp.int32)

    def step(_, lohi):
        lo, hi = lohi
        mid = lo + (hi - lo + 1) // 2
        cnt = jnp.sum((bits >= mid).astype(jnp.int32), axis=(1, 2),
                      keepdims=True)
        ge = cnt >= _K
        return jnp.where(ge, mid, lo), jnp.where(ge, hi, mid - 1)

    lo0 = jnp.zeros((_B, 1, 1), jnp.int32)
    hi0 = jnp.full((_B, 1, 1), 0x7F800000, jnp.int32)
    lo, _ = jax.lax.fori_loop(0, 31, step, (lo0, hi0))
    comp = bits >= lo
    wta = jnp.where(comp, sig, jnp.float32(0.0))
    p = jax.nn.sigmoid(_SLOPE * wta)
    hard = (p > 0.5).astype(jnp.float32)
    bm = p + (hard - p)
    bm_ref[...] = bm.reshape(_B, 1, _HM, _WM)
    reps_w = _IMG // _WM
    reps_h = _IMG // _HM
    a = jnp.broadcast_to(bm[:, :, None, :], (_B, _HM, reps_w, _WM))
    a = a.reshape(_B, _HM, _IMG)
    adj = jnp.broadcast_to(a[:, None, :, :], (_B, reps_h, _HM, _IMG))
    adj_ref[...] = adj.reshape(_B, _IMG, _IMG)


def _conv_kernel(xT_ref, adj_ref, wl1_ref, wl2_ref, out_ref,
                 mx, k1, xbig, oslab, yslab, insem, outsem):
    b = pl.program_id(0)
    cp_in = pltpu.make_async_copy(xT_ref.at[b], mx.at[pl.ds(2, _IMG)], insem)
    cp_in.start()
    zrow = jnp.zeros((2, _C, _IMG), jnp.bfloat16)
    mx[0:2] = zrow
    mx[_IMG + 2:_HPAD] = zrow
    k1[0:2] = zrow
    k1[_IMG + 2:_HPAD] = zrow
    cp_in.wait()

    def mask_chunk(i, _):
        rows = adj_ref[0, pl.ds(i * _TH, _TH), :].astype(jnp.bfloat16)
        mx[pl.ds(2 + i * _TH, _TH)] = (
            mx[pl.ds(2 + i * _TH, _TH)] * rows[:, None, :])
        return 0
    jax.lax.fori_loop(0, _NS, mask_chunk, 0)

    def conv_pass(src, wl_ref, is_pass2):
        wl = wl_ref[...]

        def slab_body(s, _):
            h0 = s * _TH
            for dw in range(_KW):
                sh = 2 - dw
                c0 = dw * _C
                if sh > 0:
                    xbig[:, c0:c0 + _C, sh:] = (
                        src[pl.ds(h0, _TH + 4), :, :_IMG - sh])
                    xbig[:, c0:c0 + _C, :sh] = jnp.zeros(
                        (_TH + 4, _C, sh), jnp.bfloat16)
                elif sh < 0:
                    xbig[:, c0:c0 + _C, :sh] = (
                        src[pl.ds(h0, _TH + 4), :, -sh:])
                    xbig[:, c0:c0 + _C, sh:] = jnp.zeros(
                        (_TH + 4, _C, -sh), jnp.bfloat16)
                else:
                    xbig[:, c0:c0 + _C, :] = src[pl.ds(h0, _TH + 4)]

            def row_body(rq, _):
                for k in range(_RU):
                    r = rq * _RU + k
                    xs = xbig[pl.ds(r, _KH)].reshape(_KDIM, _IMG)
                    yslab[r] = jax.lax.dot_general(
                        wl, xs, (((1,), (0,)), ((), ())),
                        preferred_element_type=jnp.float32)
                return 0
            jax.lax.fori_loop(0, _TH // _RU, row_body, 0)

            mrows = adj_ref[0, pl.ds(h0, _TH), :][:, None, :] == 1.0
            mxs = src[pl.ds(h0 + 2, _TH)]
            if is_pass2:
                oslab[...] = jnp.where(
                    mrows, mxs.astype(jnp.float32), yslab[...])
                cp_out = pltpu.make_async_copy(
                    oslab, out_ref.at[b, pl.ds(h0, _TH)], outsem)
                cp_out.start()
                cp_out.wait()
            else:
                k1[pl.ds(h0 + 2, _TH)] = jnp.where(
                    mrows, mxs, yslab[...].astype(jnp.bfloat16))
            return 0
        jax.lax.fori_loop(0, _NS, slab_body, 0)

    conv_pass(mx, wl1_ref, False)
    conv_pass(k1, wl2_ref, True)


def _build_wl(wr, wi):
    wrz = wr.at[:, :, _KH // 2, _KW // 2].set(0.0)
    wiz = wi.at[:, :, _KH // 2, _KW // 2].set(0.0)
    top = jnp.concatenate([wrz, -wiz], axis=1)
    bot = jnp.concatenate([wiz, wrz], axis=1)
    wfull = jnp.concatenate([top, bot], axis=0)
    return wfull.transpose(0, 2, 3, 1).reshape(_C, _KDIM)


def kernel(x, my_input_1, conv_mask_w, wr0, wi0, wr1, wi1):
    bm, adj = pl.pallas_call(
        _mask_kernel,
        out_shape=(
            jax.ShapeDtypeStruct((_B, 1, _HM, _WM), jnp.float32),
            jax.ShapeDtypeStruct((_B, _IMG, _IMG), jnp.float32),
        ),
    )(my_input_1.reshape(_B, 1), conv_mask_w.reshape(_HM, _WM))

    wl1 = _build_wl(wr0, wi0).astype(jnp.bfloat16)
    wl2 = _build_wl(wr1, wi1).astype(jnp.bfloat16)
    xc = jnp.concatenate([x[..., 0], x[..., 1]], axis=1)
    xT = xc.transpose(0, 2, 1, 3).astype(jnp.bfloat16)

    k2T = pl.pallas_call(
        _conv_kernel,
        grid=(_B,),
        in_specs=[
            pl.BlockSpec(memory_space=pl.ANY),
            pl.BlockSpec((1, _IMG, _IMG), lambda b: (b, 0, 0)),
            pl.BlockSpec((_C, _KDIM), lambda b: (0, 0)),
            pl.BlockSpec((_C, _KDIM), lambda b: (0, 0)),
        ],
        out_specs=pl.BlockSpec(memory_space=pl.ANY),
        out_shape=jax.ShapeDtypeStruct((_B, _IMG, _C, _IMG), jnp.float32),
        scratch_shapes=[
            pltpu.VMEM((_HPAD, _C, _IMG), jnp.bfloat16),
            pltpu.VMEM((_HPAD, _C, _IMG), jnp.bfloat16),
            pltpu.VMEM((_TH + 4, _KW * _C, _IMG), jnp.bfloat16),
            pltpu.VMEM((_TH, _C, _IMG), jnp.float32),
            pltpu.VMEM((_TH, _C, _IMG), jnp.float32),
            pltpu.SemaphoreType.DMA,
            pltpu.SemaphoreType.DMA,
        ],
    )(xT, adj, wl1, wl2)

    kc = k2T.transpose(0, 2, 1, 3)
    kspace_pred = jnp.stack([kc[:, :_NC], kc[:, _NC:]], axis=-1)
    return kspace_pred, adj.reshape(_B, 1, _IMG, _IMG), bm


# speedup vs baseline: 1.0572x; 1.0554x over previous
import jax
import jax.numpy as jnp
from jax.experimental import pallas as pl
from jax.experimental.pallas import tpu as pltpu

_B = 4
_NC = 16
_IMG = 320
_HM = 32
_WM = 32
_K = 256
_ACS = 8
_SLOPE = 5.0
_KH = 5
_KW = 5
_C = 2 * _NC
_TH = 32
_NS = _IMG // _TH
_HPAD = _IMG + 4
_KDIM = _KH * _KW * _C
_RU = 16


def _mask_kernel(mi_ref, cw_ref, bm_ref, adj_ref):
    w = cw_ref[...]
    col = jax.lax.broadcasted_iota(jnp.int32, (_HM, _WM), 1)
    acs_lo = _WM // 2 - _ACS // 2
    acs_hi = _WM // 2 + _ACS // 2 + 1
    w = jnp.where((col >= acs_lo) & (col < acs_hi), jnp.float32(1.0e7), w)
    mi = mi_ref[...].reshape(_B, 1, 1)
    sig = jax.nn.sigmoid(mi * w[None, :, :])
    bits = jax.lax.bitcast_convert_type(sig, jnp.int32)

    def step(_, lohi):
        lo, hi = lohi
        mid = lo + (hi - lo + 1) // 2
        cnt = jnp.sum((bits >= mid).astype(jnp.int32), axis=(1, 2),
                      keepdims=True)
        ge = cnt >= _K
        return jnp.where(ge, mid, lo), jnp.where(ge, hi, mid - 1)

    lo0 = jnp.zeros((_B, 1, 1), jnp.int32)
    hi0 = jnp.full((_B, 1, 1), 0x7F800000, jnp.int32)
    lo, _ = jax.lax.fori_loop(0, 31, step, (lo0, hi0))
    comp = bits >= lo
    wta = jnp.where(comp, sig, jnp.float32(0.0))
    p = jax.nn.sigmoid(_SLOPE * wta)
    hard = (p > 0.5).astype(jnp.float32)
    bm = p + (hard - p)
    bm_ref[...] = bm.reshape(_B, 1, _HM, _WM)
    reps_w = _IMG // _WM
    reps_h = _IMG // _HM
    a = jnp.broadcast_to(bm[:, :, None, :], (_B, _HM, reps_w, _WM))
    a = a.reshape(_B, _HM, _IMG)
    adj = jnp.broadcast_to(a[:, None, :, :], (_B, reps_h, _HM, _IMG))
    adj_ref[...] = adj.reshape(_B, _IMG, _IMG)


def _conv_kernel(xT_ref, adj_ref, wl1_ref, wl2_ref, out_ref,
                 mx, k1, xbig, oslab, yslab, insem, outsem):
    b = pl.program_id(0)
    cp_in = pltpu.make_async_copy(xT_ref.at[b], mx.at[pl.ds(2, _IMG)], insem)
    cp_in.start()
    zrow = jnp.zeros((2, _C, _IMG), jnp.bfloat16)
    mx[0:2] = zrow
    mx[_IMG + 2:_HPAD] = zrow
    k1[0:2] = zrow
    k1[_IMG + 2:_HPAD] = zrow
    cp_in.wait()

    def mask_chunk(i, _):
        rows = adj_ref[0, pl.ds(i * _TH, _TH), :].astype(jnp.bfloat16)
        mx[pl.ds(2 + i * _TH, _TH)] = (
            mx[pl.ds(2 + i * _TH, _TH)] * rows[:, None, :])
        return 0
    jax.lax.fori_loop(0, _NS, mask_chunk, 0)

    def conv_pass(src, wl_ref, is_pass2):
        wl = wl_ref[...]

        def slab_body(s, _):
            h0 = s * _TH
            for dw in range(_KW):
                sh = 2 - dw
                c0 = dw * _C
                if sh > 0:
                    xbig[:, c0:c0 + _C, sh:] = (
                        src[pl.ds(h0, _TH + 4), :, :_IMG - sh])
                    xbig[:, c0:c0 + _C, :sh] = jnp.zeros(
                        (_TH + 4, _C, sh), jnp.bfloat16)
                elif sh < 0:
                    xbig[:, c0:c0 + _C, :sh] = (
                        src[pl.ds(h0, _TH + 4), :, -sh:])
                    xbig[:, c0:c0 + _C, sh:] = jnp.zeros(
                        (_TH + 4, _C, -sh), jnp.bfloat16)
                else:
                    xbig[:, c0:c0 + _C, :] = src[pl.ds(h0, _TH + 4)]

            def row_body(rq, _):
                for k in range(_RU):
                    r = rq * _RU + k
                    xs = xbig[pl.ds(r, _KH)].reshape(_KDIM, _IMG)
                    y = jax.lax.dot_general(
                        wl, xs, (((1,), (0,)), ((), ())),
                        preferred_element_type=jnp.float32)
                    h = h0 + r
                    mrow = adj_ref[0, h, :]
                    if is_pass2:
                        oslab[r] = jnp.where(
                            mrow[None, :] == 1.0,
                            src[h + 2].astype(jnp.float32), y)
                    else:
                        k1[h + 2] = jnp.where(
                            mrow[None, :] == 1.0,
                            src[h + 2], y.astype(jnp.bfloat16))
                return 0
            jax.lax.fori_loop(0, _TH // _RU, row_body, 0)

            if is_pass2:
                cp_out = pltpu.make_async_copy(
                    oslab, out_ref.at[b, pl.ds(h0, _TH)], outsem)
                cp_out.start()
                cp_out.wait()
            return 0
        jax.lax.fori_loop(0, _NS, slab_body, 0)

    conv_pass(mx, wl1_ref, False)
    conv_pass(k1, wl2_ref, True)


def _build_wl(wr, wi):
    wrz = wr.at[:, :, _KH // 2, _KW // 2].set(0.0)
    wiz = wi.at[:, :, _KH // 2, _KW // 2].set(0.0)
    top = jnp.concatenate([wrz, -wiz], axis=1)
    bot = jnp.concatenate([wiz, wrz], axis=1)
    wfull = jnp.concatenate([top, bot], axis=0)
    return wfull.transpose(0, 2, 3, 1).reshape(_C, _KDIM)


def kernel(x, my_input_1, conv_mask_w, wr0, wi0, wr1, wi1):
    bm, adj = pl.pallas_call(
        _mask_kernel,
        out_shape=(
            jax.ShapeDtypeStruct((_B, 1, _HM, _WM), jnp.float32),
            jax.ShapeDtypeStruct((_B, _IMG, _IMG), jnp.float32),
        ),
    )(my_input_1.reshape(_B, 1), conv_mask_w.reshape(_HM, _WM))

    wl1 = _build_wl(wr0, wi0).astype(jnp.bfloat16)
    wl2 = _build_wl(wr1, wi1).astype(jnp.bfloat16)
    xc = jnp.concatenate([x[..., 0], x[..., 1]], axis=1)
    xT = xc.transpose(0, 2, 1, 3).astype(jnp.bfloat16)

    k2T = pl.pallas_call(
        _conv_kernel,
        grid=(_B,),
        in_specs=[
            pl.BlockSpec(memory_space=pl.ANY),
            pl.BlockSpec((1, _IMG, _IMG), lambda b: (b, 0, 0)),
            pl.BlockSpec((_C, _KDIM), lambda b: (0, 0)),
            pl.BlockSpec((_C, _KDIM), lambda b: (0, 0)),
        ],
        out_specs=pl.BlockSpec(memory_space=pl.ANY),
        out_shape=jax.ShapeDtypeStruct((_B, _IMG, _C, _IMG), jnp.float32),
        scratch_shapes=[
            pltpu.VMEM((_HPAD, _C, _IMG), jnp.bfloat16),
            pltpu.VMEM((_HPAD, _C, _IMG), jnp.bfloat16),
            pltpu.VMEM((_TH + 4, _KW * _C, _IMG), jnp.bfloat16),
            pltpu.VMEM((_TH, _C, _IMG), jnp.float32),
            pltpu.VMEM((_TH, _C, _IMG), jnp.float32),
            pltpu.SemaphoreType.DMA,
            pltpu.SemaphoreType.DMA,
        ],
    )(xT, adj, wl1, wl2)

    kc = k2T.transpose(0, 2, 1, 3)
    kspace_pred = jnp.stack([kc[:, :_NC], kc[:, _NC:]], axis=-1)
    return kspace_pred, adj.reshape(_B, 1, _IMG, _IMG), bm


# speedup vs baseline: 1.2379x; 1.1709x over previous
import jax
import jax.numpy as jnp
from jax.experimental import pallas as pl
from jax.experimental.pallas import tpu as pltpu

_B = 4
_NC = 16
_IMG = 320
_HM = 32
_WM = 32
_K = 256
_ACS = 8
_SLOPE = 5.0
_KH = 5
_KW = 5
_C = 2 * _NC
_TH = 32
_NS = _IMG // _TH
_HPAD = _IMG + 4
_KDIM = _KH * _KW * _C
_RU = 32


def _mask_kernel(mi_ref, cw_ref, bm_ref, adj_ref):
    w = cw_ref[...]
    col = jax.lax.broadcasted_iota(jnp.int32, (_HM, _WM), 1)
    acs_lo = _WM // 2 - _ACS // 2
    acs_hi = _WM // 2 + _ACS // 2 + 1
    w = jnp.where((col >= acs_lo) & (col < acs_hi), jnp.float32(1.0e7), w)
    mi = mi_ref[...].reshape(_B, 1, 1)
    sig = jax.nn.sigmoid(mi * w[None, :, :])
    bits = jax.lax.bitcast_convert_type(sig, jnp.int32)

    def step(_, lohi):
        lo, hi = lohi
        mid = lo + (hi - lo + 1) // 2
        cnt = jnp.sum((bits >= mid).astype(jnp.int32), axis=(1, 2),
                      keepdims=True)
        ge = cnt >= _K
        return jnp.where(ge, mid, lo), jnp.where(ge, hi, mid - 1)

    lo0 = jnp.zeros((_B, 1, 1), jnp.int32)
    hi0 = jnp.full((_B, 1, 1), 0x7F800000, jnp.int32)
    lo, _ = jax.lax.fori_loop(0, 31, step, (lo0, hi0))
    comp = bits >= lo
    wta = jnp.where(comp, sig, jnp.float32(0.0))
    p = jax.nn.sigmoid(_SLOPE * wta)
    hard = (p > 0.5).astype(jnp.float32)
    bm = p + (hard - p)
    bm_ref[...] = bm.reshape(_B, 1, _HM, _WM)
    reps_w = _IMG // _WM
    reps_h = _IMG // _HM
    a = jnp.broadcast_to(bm[:, :, None, :], (_B, _HM, reps_w, _WM))
    a = a.reshape(_B, _HM, _IMG)
    adj = jnp.broadcast_to(a[:, None, :, :], (_B, reps_h, _HM, _IMG))
    adj_ref[...] = adj.reshape(_B, _IMG, _IMG)


def _conv_kernel(xT_ref, adj_ref, wl1_ref, wl2_ref, out_ref,
                 mx, k1, xbig, oslab, yslab, insem, outsem):
    b = pl.program_id(0)
    cp_in = pltpu.make_async_copy(xT_ref.at[b], mx.at[pl.ds(2, _IMG)], insem)
    cp_in.start()
    zrow = jnp.zeros((2, _C, _IMG), jnp.bfloat16)
    mx[0:2] = zrow
    mx[_IMG + 2:_HPAD] = zrow
    k1[0:2] = zrow
    k1[_IMG + 2:_HPAD] = zrow
    cp_in.wait()

    def mask_chunk(i, _):
        rows = adj_ref[0, pl.ds(i * _TH, _TH), :].astype(jnp.bfloat16)
        mx[pl.ds(2 + i * _TH, _TH)] = (
            mx[pl.ds(2 + i * _TH, _TH)] * rows[:, None, :])
        return 0
    jax.lax.fori_loop(0, _NS, mask_chunk, 0)

    def conv_pass(src, wl_ref, is_pass2):
        wl = wl_ref[...]

        def slab_body(s, _):
            h0 = s * _TH
            for dw in range(_KW):
                sh = 2 - dw
                c0 = dw * _C
                if sh > 0:
                    xbig[:, c0:c0 + _C, sh:] = (
                        src[pl.ds(h0, _TH + 4), :, :_IMG - sh])
                    xbig[:, c0:c0 + _C, :sh] = jnp.zeros(
                        (_TH + 4, _C, sh), jnp.bfloat16)
                elif sh < 0:
                    xbig[:, c0:c0 + _C, :sh] = (
                        src[pl.ds(h0, _TH + 4), :, -sh:])
                    xbig[:, c0:c0 + _C, sh:] = jnp.zeros(
                        (_TH + 4, _C, -sh), jnp.bfloat16)
                else:
                    xbig[:, c0:c0 + _C, :] = src[pl.ds(h0, _TH + 4)]

            def row_body(rq, _):
                for k in range(_RU):
                    r = rq * _RU + k
                    xs = xbig[pl.ds(r, _KH)].reshape(_KDIM, _IMG)
                    y = jax.lax.dot_general(
                        wl, xs, (((1,), (0,)), ((), ())),
                        preferred_element_type=jnp.float32)
                    h = h0 + r
                    mrow = adj_ref[0, h, :]
                    if is_pass2:
                        oslab[r] = jnp.where(
                            mrow[None, :] == 1.0,
                            src[h + 2].astype(jnp.float32), y)
                    else:
                        k1[h + 2] = jnp.where(
                            mrow[None, :] == 1.0,
                            src[h + 2], y.astype(jnp.bfloat16))
                return 0
            jax.lax.fori_loop(0, _TH // _RU, row_body, 0)

            if is_pass2:
                cp_out = pltpu.make_async_copy(
                    oslab, out_ref.at[b, pl.ds(h0, _TH)], outsem)
                cp_out.start()
                cp_out.wait()
            return 0
        jax.lax.fori_loop(0, _NS, slab_body, 0)

    conv_pass(mx, wl1_ref, False)
    conv_pass(k1, wl2_ref, True)


def _build_wl(wr, wi):
    wrz = wr.at[:, :, _KH // 2, _KW // 2].set(0.0)
    wiz = wi.at[:, :, _KH // 2, _KW // 2].set(0.0)
    top = jnp.concatenate([wrz, -wiz], axis=1)
    bot = jnp.concatenate([wiz, wrz], axis=1)
    wfull = jnp.concatenate([top, bot], axis=0)
    return wfull.transpose(0, 2, 3, 1).reshape(_C, _KDIM)


def kernel(x, my_input_1, conv_mask_w, wr0, wi0, wr1, wi1):
    bm, adj = pl.pallas_call(
        _mask_kernel,
        out_shape=(
            jax.ShapeDtypeStruct((_B, 1, _HM, _WM), jnp.float32),
            jax.ShapeDtypeStruct((_B, _IMG, _IMG), jnp.float32),
        ),
    )(my_input_1.reshape(_B, 1), conv_mask_w.reshape(_HM, _WM))

    wl1 = _build_wl(wr0, wi0).astype(jnp.bfloat16)
    wl2 = _build_wl(wr1, wi1).astype(jnp.bfloat16)
    xc = jnp.concatenate([x[..., 0], x[..., 1]], axis=1)
    xT = xc.transpose(0, 2, 1, 3).astype(jnp.bfloat16)

    k2T = pl.pallas_call(
        _conv_kernel,
        grid=(_B,),
        in_specs=[
            pl.BlockSpec(memory_space=pl.ANY),
            pl.BlockSpec((1, _IMG, _IMG), lambda b: (b, 0, 0)),
            pl.BlockSpec((_C, _KDIM), lambda b: (0, 0)),
            pl.BlockSpec((_C, _KDIM), lambda b: (0, 0)),
        ],
        out_specs=pl.BlockSpec(memory_space=pl.ANY),
        out_shape=jax.ShapeDtypeStruct((_B, _IMG, _C, _IMG), jnp.float32),
        scratch_shapes=[
            pltpu.VMEM((_HPAD, _C, _IMG), jnp.bfloat16),
            pltpu.VMEM((_HPAD, _C, _IMG), jnp.bfloat16),
            pltpu.VMEM((_TH + 4, _KW * _C, _IMG), jnp.bfloat16),
            pltpu.VMEM((_TH, _C, _IMG), jnp.float32),
            pltpu.VMEM((_TH, _C, _IMG), jnp.float32),
            pltpu.SemaphoreType.DMA,
            pltpu.SemaphoreType.DMA,
        ],
    )(xT, adj, wl1, wl2)

    kc = k2T.transpose(0, 2, 1, 3)
    kspace_pred = jnp.stack([kc[:, :_NC], kc[:, _NC:]], axis=-1)
    return kspace_pred, adj.reshape(_B, 1, _IMG, _IMG), bm
